# baseline (speedup 1.0000x reference)
import jax
import jax.numpy as jnp
from jax import lax
from jax.experimental import pallas as pl
from jax.experimental.pallas import tpu as pltpu
from jax.experimental.pallas import tpu_sc as plsc

B = 8
N = 5000
NS = 512
L = 16
NCH = NS // L
NPART = 4
ROWS_PER_PART = NS // NPART
BIG = 1e30
SENT = 4e30
IBIG = 2**30


def _splat_i(x):
    return jnp.broadcast_to(jnp.asarray(x, jnp.int32), (L,))


def _splat_f(x):
    return jnp.broadcast_to(jnp.asarray(x, jnp.float32), (L,))


def _lane0():
    return lax.iota(jnp.int32, L) == 0


def _emd_body(s1_hbm, s2_hbm, idx_hbm, out_hbm,
              stage, idx_v, ax, ay, az, bx, by, bz,
              rv, ri, rv2, ri2, rowpen, colpen, cmin, selbuf, outrow,
              rv_sh, ri_sh):
    c = lax.axis_index("c")
    s = lax.axis_index("s")
    bl = s % NPART
    part = s // NPART
    b = c * NPART + bl
    iota = lax.iota(jnp.int32, L)

    pltpu.sync_copy(idx_hbm.at[b], idx_v)

    def gather_pts(src_hbm, dx_ref, dy_ref, dz_ref):
        pltpu.sync_copy(src_hbm.at[b], stage)

        def g(k, _):
            flat = idx_v[pl.ds(k * L, L)] * 3
            dx_ref[pl.ds(k * L, L)] = plsc.load_gather(stage, [flat])
            dy_ref[pl.ds(k * L, L)] = plsc.load_gather(
                stage, [flat + _splat_i(1)])
            dz_ref[pl.ds(k * L, L)] = plsc.load_gather(
                stage, [flat + _splat_i(2)])
            return 0

        lax.fori_loop(0, NCH, g, 0)

    gather_pts(s1_hbm, ax, ay, az)
    gather_pts(s2_hbm, bx, by, bz)

    def z(k, _):
        rowpen[pl.ds(k * L, L)] = _splat_f(0.0)
        colpen[pl.ds(k * L, L)] = _splat_f(0.0)
        return 0

    lax.fori_loop(0, NCH, z, 0)

    def row_min2(i_splat):
        aix = plsc.load_gather(ax, [i_splat])
        aiy = plsc.load_gather(ay, [i_splat])
        aiz = plsc.load_gather(az, [i_splat])

        def scan(j, carry):
            v1, i1, v2, i2 = carry
            sl = pl.ds(j * L, L)
            dx = aix - bx[sl]
            dy = aiy - by[sl]
            dz = aiz - bz[sl]
            d2 = dx * dx + dy * dy + dz * dz + colpen[sl]
            ids = j * L + iota
            lt1 = d2 < v1
            lt2 = d2 < v2
            nv2 = jnp.where(lt1, v1, jnp.where(lt2, d2, v2))
            ni2 = jnp.where(lt1, i1, jnp.where(lt2, ids, i2))
            v1 = jnp.where(lt1, d2, v1)
            i1 = jnp.where(lt1, ids, i1)
            return v1, i1, nv2, ni2

        v1, i1, v2, i2 = lax.fori_loop(
            0, NCH, scan,
            (_splat_f(SENT), _splat_i(0), _splat_f(SENT), _splat_i(0)),
            unroll=4)
        mval = jnp.min(v1)
        midx = jnp.min(jnp.where(v1 == mval, i1, IBIG))
        donor = iota == (midx & (L - 1))
        w = jnp.where(donor, v2, v1)
        wi = jnp.where(donor, i2, i1)
        m2val = jnp.min(w)
        m2idx = jnp.min(jnp.where(w == m2val, wi, IBIG))
        return mval, midx, m2val, m2idx

    def store1(ref, i_splat, val_splat):
        plsc.store_scatter(ref, [i_splat], val_splat, mask=_lane0())

    def build(i, _):
        i_splat = _splat_i(i)
        mval, midx, m2val, m2idx = row_min2(i_splat)
        store1(rv, i_splat, _splat_f(mval))
        store1(ri, i_splat, _splat_i(midx))
        store1(rv2, i_splat, _splat_f(m2val))
        store1(ri2, i_splat, _splat_i(m2idx))
        return 0

    lax.fori_loop(part * ROWS_PER_PART, (part + 1) * ROWS_PER_PART, build, 0)

    rsl = pl.ds(part * ROWS_PER_PART, ROWS_PER_PART)
    pltpu.sync_copy(rv.at[rsl], rv_sh.at[bl, 0, rsl])
    pltpu.sync_copy(rv2.at[rsl], rv_sh.at[bl, 1, rsl])
    pltpu.sync_copy(ri.at[rsl], ri_sh.at[bl, 0, rsl])
    pltpu.sync_copy(ri2.at[rsl], ri_sh.at[bl, 1, rsl])
    plsc.subcore_barrier()

    @pl.when(s < NPART)
    def _greedy():
        pltpu.sync_copy(rv_sh.at[bl, 0], rv)
        pltpu.sync_copy(rv_sh.at[bl, 1], rv2)
        pltpu.sync_copy(ri_sh.at[bl, 0], ri)
        pltpu.sync_copy(ri_sh.at[bl, 1], ri2)

        def upd_cmin(j):
            sl = pl.ds(j * L, L)
            m = jnp.min(rv[sl] + rowpen[sl])
            store1(cmin, _splat_i(j), _splat_f(m))

        def init_cmin(j, _):
            upd_cmin(j)
            return 0

        lax.fori_loop(0, NCH, init_cmin, 0)

        def step(t, _):
            def not_done(carry):
                return carry[0] == 0

            def pop(carry):
                m0 = cmin[pl.ds(0, L)]
                m1 = cmin[pl.ds(L, L)]
                upd = m1 < m0
                minv = jnp.minimum(m0, m1)
                mini = jnp.where(upd, iota + L, iota)
                cm = jnp.min(minv)
                jb = jnp.min(jnp.where(minv == cm, mini, IBIG))
                sl = pl.ds(jb * L, L)
                chunk = rv[sl] + rowpen[sl]
                lane = jnp.min(jnp.where(chunk == cm, iota, IBIG))
                rs = jb * L + lane
                r_splat = _splat_i(rs)
                csvec = plsc.load_gather(ri, [r_splat])
                cp = jnp.min(plsc.load_gather(colpen, [csvec]))

                def stale(_):
                    c2vec = plsc.load_gather(ri2, [r_splat])
                    v2vec = plsc.load_gather(rv2, [r_splat])
                    cp2vec = plsc.load_gather(colpen, [c2vec])
                    q = jnp.min(v2vec + cp2vec)

                    def fallback(_):
                        store1(rv, r_splat, v2vec)
                        store1(ri, r_splat, c2vec)
                        store1(rv2, r_splat, _splat_f(SENT))
                        return 0

                    def rescan(_):
                        nv, ni, nv2, ni2 = row_min2(r_splat)
                        store1(rv, r_splat, _splat_f(nv))
                        store1(ri, r_splat, _splat_i(ni))
                        store1(rv2, r_splat, _splat_f(nv2))
                        store1(ri2, r_splat, _splat_i(ni2))
                        return 0

                    lax.cond(q < 1e29, fallback, rescan, 0)
                    upd_cmin(jb)
                    return jnp.int32(0)

                def take(_):
                    return jnp.int32(1)

                done = lax.cond(cp == 0.0, take, stale, 0)
                return done, rs, jb, cm

            _, rsel, jbest, mval = lax.while_loop(
                not_done, pop,
                (jnp.int32(0), jnp.int32(0), jnp.int32(0), jnp.float32(0.0)))

            r_splat = _splat_i(rsel)
            csel = jnp.min(plsc.load_gather(ri, [r_splat]))
            store1(selbuf, _splat_i(t), _splat_f(mval))
            store1(rowpen, r_splat, _splat_f(BIG))
            store1(colpen, _splat_i(csel), _splat_f(BIG))
            upd_cmin(jbest)
            return 0

        lax.fori_loop(0, NS, step, 0)

        def fin(j, acc):
            x = selbuf[pl.ds(j * L, L)]
            y = plsc.bitcast(
                lax.shift_right_logical(plsc.bitcast(x, jnp.int32),
                                        _splat_i(1)) + _splat_i(0x1FBD1DF5),
                jnp.float32)
            y = 0.5 * (y + x / y)
            y = 0.5 * (y + x / y)
            y = 0.5 * (y + x / y)
            return acc + y

        acc = lax.fori_loop(0, NCH, fin, _splat_f(0.0), unroll=2)
        outrow[...] = acc
        pltpu.sync_copy(outrow, out_hbm.at[b])


@jax.jit
def _emd_call(S1, S2, idx):
    mesh = plsc.VectorSubcoreMesh(core_axis_name="c", subcore_axis_name="s")
    kern = pl.kernel(
        _emd_body,
        out_type=jax.ShapeDtypeStruct((B, L), jnp.float32),
        mesh=mesh,
        compiler_params=pltpu.CompilerParams(
            use_tc_tiling_on_sc=False, needs_layout_passes=False),
        scratch_types=[
            pltpu.VMEM((N * 3,), jnp.float32),
            pltpu.VMEM((NS,), jnp.int32),
            pltpu.VMEM((NS,), jnp.float32),
            pltpu.VMEM((NS,), jnp.float32),
            pltpu.VMEM((NS,), jnp.float32),
            pltpu.VMEM((NS,), jnp.float32),
            pltpu.VMEM((NS,), jnp.float32),
            pltpu.VMEM((NS,), jnp.float32),
            pltpu.VMEM((NS,), jnp.float32),
            pltpu.VMEM((NS,), jnp.int32),
            pltpu.VMEM((NS,), jnp.float32),
            pltpu.VMEM((NS,), jnp.int32),
            pltpu.VMEM((NS,), jnp.float32),
            pltpu.VMEM((NS,), jnp.float32),
            pltpu.VMEM((NCH,), jnp.float32),
            pltpu.VMEM((NS,), jnp.float32),
            pltpu.VMEM((L,), jnp.float32),
            pltpu.VMEM_SHARED((NPART, 2, NS), jnp.float32),
            pltpu.VMEM_SHARED((NPART, 2, NS), jnp.int32),
        ],
    )
    return kern(S1, S2, idx)


def kernel(S1, S2):
    idx = jnp.stack([
        jax.random.permutation(
            jax.random.fold_in(jax.random.key(42), i), N)[:NS]
        for i in range(B)
    ]).astype(jnp.int32)
    out = _emd_call(S1.reshape(B, N * 3), S2.reshape(B, N * 3), idx)
    return jnp.sum(out) / jnp.float32(NS * B)


# speedup vs baseline: 1.0375x; 1.0375x over previous
import jax
import jax.numpy as jnp
from jax import lax
from jax.experimental import pallas as pl
from jax.experimental.pallas import tpu as pltpu
from jax.experimental.pallas import tpu_sc as plsc

B = 8
N = 5000
NS = 512
L = 16
NCH = NS // L
NPART = 4
ROWS_PER_PART = NS // NPART
BIG = 1e30
IBIG = 2**30


def _splat_i(x):
    return jnp.broadcast_to(jnp.asarray(x, jnp.int32), (L,))


def _splat_f(x):
    return jnp.broadcast_to(jnp.asarray(x, jnp.float32), (L,))


def _lane0():
    return lax.iota(jnp.int32, L) == 0


def _emd_body(s1_hbm, s2_hbm, idx_hbm, out_hbm,
              stage, idx_v, ax, ay, az, bx, by, bz,
              rv, ri, rowpen, colpen, cmin, selbuf, outrow,
              rv_sh, ri_sh):
    c = lax.axis_index("c")
    s = lax.axis_index("s")
    bl = s % NPART
    part = s // NPART
    b = c * NPART + bl
    iota = lax.iota(jnp.int32, L)

    pltpu.sync_copy(idx_hbm.at[b], idx_v)

    def gather_pts(src_hbm, dx_ref, dy_ref, dz_ref):
        pltpu.sync_copy(src_hbm.at[b], stage)

        def g(k, _):
            flat = idx_v[pl.ds(k * L, L)] * 3
            dx_ref[pl.ds(k * L, L)] = plsc.load_gather(stage, [flat])
            dy_ref[pl.ds(k * L, L)] = plsc.load_gather(
                stage, [flat + _splat_i(1)])
            dz_ref[pl.ds(k * L, L)] = plsc.load_gather(
                stage, [flat + _splat_i(2)])
            return 0

        lax.fori_loop(0, NCH, g, 0)

    gather_pts(s1_hbm, ax, ay, az)
    gather_pts(s2_hbm, bx, by, bz)

    def z(k, _):
        rowpen[pl.ds(k * L, L)] = _splat_f(0.0)
        colpen[pl.ds(k * L, L)] = _splat_f(0.0)
        return 0

    lax.fori_loop(0, NCH, z, 0)

    def row_min(i_splat):
        aix = plsc.load_gather(ax, [i_splat])
        aiy = plsc.load_gather(ay, [i_splat])
        aiz = plsc.load_gather(az, [i_splat])

        def scan(j, carry):
            minv, mini = carry
            sl = pl.ds(j * L, L)
            dx = aix - bx[sl]
            dy = aiy - by[sl]
            dz = aiz - bz[sl]
            d2 = dx * dx + dy * dy + dz * dz + colpen[sl]
            upd = d2 < minv
            minv = jnp.minimum(minv, d2)
            mini = jnp.where(upd, j * L + iota, mini)
            return minv, mini

        minv, mini = lax.fori_loop(
            0, NCH, scan, (_splat_f(BIG * 4.0), _splat_i(0)), unroll=4)
        mval = jnp.min(minv)
        midx = jnp.min(jnp.where(minv == mval, mini, IBIG))
        return mval, midx

    def store1(ref, i_splat, val_splat):
        plsc.store_scatter(ref, [i_splat], val_splat, mask=_lane0())

    def build(i, _):
        i_splat = _splat_i(i)
        mval, midx = row_min(i_splat)
        store1(rv, i_splat, _splat_f(mval))
        store1(ri, i_splat, _splat_i(midx))
        return 0

    lax.fori_loop(part * ROWS_PER_PART, (part + 1) * ROWS_PER_PART, build, 0)

    rsl = pl.ds(part * ROWS_PER_PART, ROWS_PER_PART)
    pltpu.sync_copy(rv.at[rsl], rv_sh.at[bl, rsl])
    pltpu.sync_copy(ri.at[rsl], ri_sh.at[bl, rsl])
    plsc.subcore_barrier()

    @pl.when(s < NPART)
    def _greedy():
        pltpu.sync_copy(rv_sh.at[bl], rv)
        pltpu.sync_copy(ri_sh.at[bl], ri)

        def upd_cmin(j):
            sl = pl.ds(j * L, L)
            m = jnp.min(rv[sl] + rowpen[sl])
            store1(cmin, _splat_i(j), _splat_f(m))

        def init_cmin(j, _):
            upd_cmin(j)
            return 0

        lax.fori_loop(0, NCH, init_cmin, 0)

        def step(t, _):
            def not_done(carry):
                return carry[0] == 0

            def pop(carry):
                m0 = cmin[pl.ds(0, L)]
                m1 = cmin[pl.ds(L, L)]
                upd = m1 < m0
                minv = jnp.minimum(m0, m1)
                mini = jnp.where(upd, iota + L, iota)
                cm = jnp.min(minv)
                jb = jnp.min(jnp.where(minv == cm, mini, IBIG))
                sl = pl.ds(jb * L, L)
                chunk = rv[sl] + rowpen[sl]
                lane = jnp.min(jnp.where(chunk == cm, iota, IBIG))
                rs = jb * L + lane
                r_splat = _splat_i(rs)
                csvec = plsc.load_gather(ri, [r_splat])
                cp = jnp.min(plsc.load_gather(colpen, [csvec]))

                def rescan(_):
                    nv, ni = row_min(r_splat)
                    store1(rv, r_splat, _splat_f(nv))
                    store1(ri, r_splat, _splat_i(ni))
                    upd_cmin(jb)
                    return jnp.int32(0)

                def take(_):
                    return jnp.int32(1)

                done = lax.cond(cp == 0.0, take, rescan, 0)
                return done, rs, jb, cm

            _, rsel, jbest, mval = lax.while_loop(
                not_done, pop,
                (jnp.int32(0), jnp.int32(0), jnp.int32(0), jnp.float32(0.0)))

            r_splat = _splat_i(rsel)
            csel = jnp.min(plsc.load_gather(ri, [r_splat]))
            store1(selbuf, _splat_i(t), _splat_f(mval))
            store1(rowpen, r_splat, _splat_f(BIG))
            store1(colpen, _splat_i(csel), _splat_f(BIG))
            upd_cmin(jbest)
            return 0

        lax.fori_loop(0, NS, step, 0)

        def fin(j, acc):
            x = selbuf[pl.ds(j * L, L)]
            y = plsc.bitcast(
                lax.shift_right_logical(plsc.bitcast(x, jnp.int32),
                                        _splat_i(1)) + _splat_i(0x1FBD1DF5),
                jnp.float32)
            y = 0.5 * (y + x / y)
            y = 0.5 * (y + x / y)
            y = 0.5 * (y + x / y)
            return acc + y

        acc = lax.fori_loop(0, NCH, fin, _splat_f(0.0), unroll=2)
        outrow[...] = acc
        pltpu.sync_copy(outrow, out_hbm.at[b])


@jax.jit
def _emd_call(S1, S2, idx):
    mesh = plsc.VectorSubcoreMesh(core_axis_name="c", subcore_axis_name="s")
    kern = pl.kernel(
        _emd_body,
        out_type=jax.ShapeDtypeStruct((B, L), jnp.float32),
        mesh=mesh,
        compiler_params=pltpu.CompilerParams(
            use_tc_tiling_on_sc=False, needs_layout_passes=False),
        scratch_types=[
            pltpu.VMEM((N * 3,), jnp.float32),
            pltpu.VMEM((NS,), jnp.int32),
            pltpu.VMEM((NS,), jnp.float32),
            pltpu.VMEM((NS,), jnp.float32),
            pltpu.VMEM((NS,), jnp.float32),
            pltpu.VMEM((NS,), jnp.float32),
            pltpu.VMEM((NS,), jnp.float32),
            pltpu.VMEM((NS,), jnp.float32),
            pltpu.VMEM((NS,), jnp.float32),
            pltpu.VMEM((NS,), jnp.int32),
            pltpu.VMEM((NS,), jnp.float32),
            pltpu.VMEM((NS,), jnp.float32),
            pltpu.VMEM((NCH,), jnp.float32),
            pltpu.VMEM((NS,), jnp.float32),
            pltpu.VMEM((L,), jnp.float32),
            pltpu.VMEM_SHARED((NPART, NS), jnp.float32),
            pltpu.VMEM_SHARED((NPART, NS), jnp.int32),
        ],
    )
    return kern(S1, S2, idx)


def kernel(S1, S2):
    idx = jnp.stack([
        jax.random.permutation(
            jax.random.fold_in(jax.random.key(42), i), N)[:NS]
        for i in range(B)
    ]).astype(jnp.int32)
    out = _emd_call(S1.reshape(B, N * 3), S2.reshape(B, N * 3), idx)
    return jnp.sum(out) / jnp.float32(NS * B)


# speedup vs baseline: 1.1645x; 1.1225x over previous
import jax
import jax.numpy as jnp
from jax import lax
from jax.experimental import pallas as pl
from jax.experimental.pallas import tpu as pltpu
from jax.experimental.pallas import tpu_sc as plsc

B = 8
N = 5000
NS = 512
L = 16
NCH = NS // L
NPART = 4
ROWS_PER_PART = NS // NPART
BIG = 1e30
IBIG = 2**30


def _splat_i(x):
    return jnp.broadcast_to(jnp.asarray(x, jnp.int32), (L,))


def _splat_f(x):
    return jnp.broadcast_to(jnp.asarray(x, jnp.float32), (L,))


def _lane0():
    return lax.iota(jnp.int32, L) == 0


def _emd_body(sx_hbm, idx_hbm, out_hbm,
              idx_v, ib0, ib1, ib2, ib3, ib4, ib5,
              ax, ay, az, bx, by, bz,
              rv, ri, rowpen, colpen, cmin, selbuf, outrow,
              rv_sh, ri_sh, sem):
    c = lax.axis_index("c")
    s = lax.axis_index("s")
    bl = s % NPART
    part = s // NPART
    b = c * NPART + bl
    iota = lax.iota(jnp.int32, L)

    pltpu.sync_copy(idx_hbm.at[b], idx_v)

    ibufs = [ib0, ib1, ib2, ib3, ib4, ib5]
    dsts = [ax, ay, az, bx, by, bz]
    for m in range(6):
        base = (m * B + b) * N

        def mk(k, _, ib=ibufs[m], base=base):
            sl = pl.ds(k * L, L)
            ib[sl] = idx_v[sl] + jnp.broadcast_to(base, (L,)).astype(jnp.int32)
            return 0

        lax.fori_loop(0, NCH, mk, 0)

    cps = [pltpu.async_copy(sx_hbm.at[ibufs[m]], dsts[m], sem)
           for m in range(6)]

    def z(k, _):
        rowpen[pl.ds(k * L, L)] = _splat_f(0.0)
        colpen[pl.ds(k * L, L)] = _splat_f(0.0)
        return 0

    lax.fori_loop(0, NCH, z, 0)

    for cp in cps:
        cp.wait()

    def row_min(i_splat):
        aix = plsc.load_gather(ax, [i_splat])
        aiy = plsc.load_gather(ay, [i_splat])
        aiz = plsc.load_gather(az, [i_splat])

        def scan(j, carry):
            minv, mini = carry
            sl = pl.ds(j * L, L)
            dx = aix - bx[sl]
            dy = aiy - by[sl]
            dz = aiz - bz[sl]
            d2 = dx * dx + dy * dy + dz * dz + colpen[sl]
            upd = d2 < minv
            minv = jnp.minimum(minv, d2)
            mini = jnp.where(upd, j * L + iota, mini)
            return minv, mini

        minv, mini = lax.fori_loop(
            0, NCH, scan, (_splat_f(BIG * 4.0), _splat_i(0)), unroll=4)
        mval = jnp.min(minv)
        midx = jnp.min(jnp.where(minv == mval, mini, IBIG))
        return mval, midx

    def store1(ref, i_splat, val_splat):
        plsc.store_scatter(ref, [i_splat], val_splat, mask=_lane0())

    def build(i, _):
        i_splat = _splat_i(i)
        mval, midx = row_min(i_splat)
        store1(rv, i_splat, _splat_f(mval))
        store1(ri, i_splat, _splat_i(midx))
        return 0

    lax.fori_loop(part * ROWS_PER_PART, (part + 1) * ROWS_PER_PART, build, 0)

    rsl = pl.ds(part * ROWS_PER_PART, ROWS_PER_PART)
    pltpu.sync_copy(rv.at[rsl], rv_sh.at[bl, rsl])
    pltpu.sync_copy(ri.at[rsl], ri_sh.at[bl, rsl])
    plsc.subcore_barrier()

    @pl.when(s < NPART)
    def _greedy():
        pltpu.sync_copy(rv_sh.at[bl], rv)
        pltpu.sync_copy(ri_sh.at[bl], ri)

        def upd_cmin(j):
            sl = pl.ds(j * L, L)
            m = jnp.min(rv[sl] + rowpen[sl])
            store1(cmin, _splat_i(j), _splat_f(m))

        def init_cmin(j, _):
            upd_cmin(j)
            return 0

        lax.fori_loop(0, NCH, init_cmin, 0)

        def step(t, _):
            def not_done(carry):
                return carry[0] == 0

            def pop(carry):
                m0 = cmin[pl.ds(0, L)]
                m1 = cmin[pl.ds(L, L)]
                upd = m1 < m0
                minv = jnp.minimum(m0, m1)
                mini = jnp.where(upd, iota + L, iota)
                cm = jnp.min(minv)
                jb = jnp.min(jnp.where(minv == cm, mini, IBIG))
                sl = pl.ds(jb * L, L)
                chunk = rv[sl] + rowpen[sl]
                lane = jnp.min(jnp.where(chunk == cm, iota, IBIG))
                rs = jb * L + lane
                r_splat = _splat_i(rs)
                csvec = plsc.load_gather(ri, [r_splat])
                cp = jnp.min(plsc.load_gather(colpen, [csvec]))

                def rescan(_):
                    nv, ni = row_min(r_splat)
                    store1(rv, r_splat, _splat_f(nv))
                    store1(ri, r_splat, _splat_i(ni))
                    upd_cmin(jb)
                    return jnp.int32(0)

                def take(_):
                    return jnp.int32(1)

                done = lax.cond(cp == 0.0, take, rescan, 0)
                return done, rs, jb, cm

            _, rsel, jbest, mval = lax.while_loop(
                not_done, pop,
                (jnp.int32(0), jnp.int32(0), jnp.int32(0), jnp.float32(0.0)))

            r_splat = _splat_i(rsel)
            csel = jnp.min(plsc.load_gather(ri, [r_splat]))
            store1(selbuf, _splat_i(t), _splat_f(mval))
            store1(rowpen, r_splat, _splat_f(BIG))
            store1(colpen, _splat_i(csel), _splat_f(BIG))
            upd_cmin(jbest)
            return 0

        lax.fori_loop(0, NS, step, 0)

        def fin(j, acc):
            x = selbuf[pl.ds(j * L, L)]
            y = plsc.bitcast(
                lax.shift_right_logical(plsc.bitcast(x, jnp.int32),
                                        _splat_i(1)) + _splat_i(0x1FBD1DF5),
                jnp.float32)
            y = 0.5 * (y + x / y)
            y = 0.5 * (y + x / y)
            y = 0.5 * (y + x / y)
            return acc + y

        acc = lax.fori_loop(0, NCH, fin, _splat_f(0.0), unroll=2)
        outrow[...] = acc
        pltpu.sync_copy(outrow, out_hbm.at[b])


@jax.jit
def _emd_call(SX, idx):
    mesh = plsc.VectorSubcoreMesh(core_axis_name="c", subcore_axis_name="s")
    kern = pl.kernel(
        _emd_body,
        out_type=jax.ShapeDtypeStruct((B, L), jnp.float32),
        mesh=mesh,
        compiler_params=pltpu.CompilerParams(
            use_tc_tiling_on_sc=False, needs_layout_passes=False),
        scratch_types=[
            pltpu.VMEM((NS,), jnp.int32),
            pltpu.VMEM((NS,), jnp.int32),
            pltpu.VMEM((NS,), jnp.int32),
            pltpu.VMEM((NS,), jnp.int32),
            pltpu.VMEM((NS,), jnp.int32),
            pltpu.VMEM((NS,), jnp.int32),
            pltpu.VMEM((NS,), jnp.int32),
            pltpu.VMEM((NS,), jnp.float32),
            pltpu.VMEM((NS,), jnp.float32),
            pltpu.VMEM((NS,), jnp.float32),
            pltpu.VMEM((NS,), jnp.float32),
            pltpu.VMEM((NS,), jnp.float32),
            pltpu.VMEM((NS,), jnp.float32),
            pltpu.VMEM((NS,), jnp.float32),
            pltpu.VMEM((NS,), jnp.int32),
            pltpu.VMEM((NS,), jnp.float32),
            pltpu.VMEM((NS,), jnp.float32),
            pltpu.VMEM((NCH,), jnp.float32),
            pltpu.VMEM((NS,), jnp.float32),
            pltpu.VMEM((L,), jnp.float32),
            pltpu.VMEM_SHARED((NPART, NS), jnp.float32),
            pltpu.VMEM_SHARED((NPART, NS), jnp.int32),
            pltpu.SemaphoreType.DMA,
        ],
    )
    return kern(SX, idx)


def kernel(S1, S2):
    idx = jnp.stack([
        jax.random.permutation(
            jax.random.fold_in(jax.random.key(42), i), N)[:NS]
        for i in range(B)
    ]).astype(jnp.int32)
    SX = jnp.concatenate([
        jnp.transpose(S1, (2, 0, 1)).reshape(-1),
        jnp.transpose(S2, (2, 0, 1)).reshape(-1),
    ])
    out = _emd_call(SX, idx)
    return jnp.sum(out) / jnp.float32(NS * B)
